# baseline (speedup 1.0000x reference)
import functools

import jax
import jax.numpy as jnp
from jax import lax
from jax.experimental import pallas as pl
from jax.experimental.pallas import tpu as pltpu
from jax.experimental.pallas import tpu_sc as plsc

N = 10000
E = 320000
D = 128
K = 16
CUTOFF = 10.0
SIGMA = 0.5

NC = 2
NS = 16
NW = NC * NS
HD = D // 2
CHUNK = 128
CH = 160
T_E = CH * CHUNK
EPT = T_E // 2
E_PAD = T_E * NS
N_PAD = 10240
ROWS_PT = N_PAD // NS

_mesh = plsc.VectorSubcoreMesh(core_axis_name="c", subcore_axis_name="s")
_sc_params = pltpu.CompilerParams(
    needs_layout_passes=False, use_tc_tiling_on_sc=False)


def _dist2_body(rT_hbm, src_hbm, dst_hbm, d2_hbm, rT_v, si_v, di_v, out_v):
    c = lax.axis_index("c")
    s = lax.axis_index("s")
    wid = c * NS + s
    base = wid * EPT
    pltpu.sync_copy(rT_hbm, rT_v)
    pltpu.sync_copy(src_hbm.at[pl.ds(base, EPT)], si_v)
    pltpu.sync_copy(dst_hbm.at[pl.ds(base, EPT)], di_v)

    def body(j, carry):
        si = si_v[pl.ds(j * 16, 16)]
        di = di_v[pl.ds(j * 16, 16)]
        dx = plsc.load_gather(rT_v, [si]) - plsc.load_gather(rT_v, [di])
        dy = plsc.load_gather(rT_v, [si + N]) - plsc.load_gather(rT_v, [di + N])
        dz = plsc.load_gather(rT_v, [si + 2 * N]) - plsc.load_gather(rT_v, [di + 2 * N])
        out_v[pl.ds(j * 16, 16)] = dx * dx + dy * dy + dz * dz
        return carry

    lax.fori_loop(0, EPT // 16, body, 0)
    pltpu.sync_copy(out_v, d2_hbm.at[pl.ds(base, EPT)])


_dist2 = functools.partial(
    pl.kernel,
    out_type=jax.ShapeDtypeStruct((E_PAD,), jnp.float32),
    mesh=_mesh,
    compiler_params=_sc_params,
    scratch_types=[
        pltpu.VMEM((3 * N,), jnp.float32),
        pltpu.VMEM((EPT,), jnp.int32),
        pltpu.VMEM((EPT,), jnp.int32),
        pltpu.VMEM((EPT,), jnp.float32),
    ],
)(_dist2_body)


BE = 1024
GB = E_PAD // BE


def _edge_filters_body(d2_ref, We_ref, wa_ref, wb_ref):
    i = pl.program_id(0)
    d2 = jnp.reshape(d2_ref[...], (1, BE))
    dist = jnp.sqrt(d2 + 1e-12)
    t = jnp.clip(dist * (1.0 / CUTOFF), 0.0, 1.0)
    env = 0.5 * (jnp.cos(jnp.float32(3.14159265358979323846) * t) + 1.0)
    rows = i * BE + lax.broadcasted_iota(jnp.int32, (1, BE), 1)
    envm = jnp.where(rows < E, env, 0.0)
    inv = 1.0 / (2.0 * SIGMA * SIGMA)
    cols = []
    for k in range(K):
        mu_k = CUTOFF * k / (K - 1)
        cols.append(jnp.exp((dist - mu_k) * (dist - mu_k) * (-inv)) * envm)
    feat = jnp.concatenate(cols, axis=0)
    dn = (((0,), (0,)), ((), ()))
    w = lax.dot_general(feat, We_ref[...], dn,
                        preferred_element_type=jnp.float32)
    wa_ref[...] = w[:, :HD]
    wb_ref[...] = w[:, HD:]


def _edge_filters(d2_3d, We):
    return pl.pallas_call(
        _edge_filters_body,
        grid=(GB,),
        in_specs=[
            pl.BlockSpec((1, 1, BE), lambda i: (i, 0, 0)),
            pl.BlockSpec((K, D), lambda i: (0, 0)),
        ],
        out_specs=[
            pl.BlockSpec((BE, HD), lambda i: (i, 0)),
            pl.BlockSpec((BE, HD), lambda i: (i, 0)),
        ],
        out_shape=[
            jax.ShapeDtypeStruct((E_PAD, HD), jnp.float32),
            jax.ShapeDtypeStruct((E_PAD, HD), jnp.float32),
        ],
    )(d2_3d, We)


def _h0_body(spin_ref, X_ref, out_ref):
    sp = spin_ref[...]
    out_ref[...] = jnp.where(sp == 0, X_ref[0:1, :], X_ref[1:2, :])


def _h0(spin2d, X):
    return pl.pallas_call(
        _h0_body,
        out_shape=jax.ShapeDtypeStruct((N_PAD, D), jnp.float32),
    )(spin2d, X)


NITER = CH // 2


def _edge_layer_body(hA_hbm, hB_hbm, wA_hbm, wB_hbm, src2_hbm, dst2_hbm,
                     zeros_hbm, outA, outB,
                     src_v, dst_v, h0_v, h1_v, w0_v, w1_v, m0_v, m1_v,
                     g0_s, g1_s, l0_s, l1_s, s0_s, s1_s, agg_sh):
    c = lax.axis_index("c")
    s = lax.axis_index("s")
    rbase = s * ROWS_PT
    pltpu.sync_copy(zeros_hbm.at[pl.ds(rbase, ROWS_PT)],
                    agg_sh.at[pl.ds(rbase, ROWS_PT)])
    pltpu.sync_copy(src2_hbm.at[pl.ds(s * CH, CH)], src_v)
    pltpu.sync_copy(dst2_hbm.at[pl.ds(s * CH, CH)], dst_v)
    plsc.subcore_barrier()

    bufs = ((h0_v, w0_v, m0_v, g0_s, l0_s, s0_s),
            (h1_v, w1_v, m1_v, g1_s, l1_s, s1_s))

    def run(h_hbm, w_hbm, out_hbm):
        def wrows(j):
            return w_hbm.at[pl.ds(s * T_E + j * CHUNK, CHUNK)]

        for b, (hb, wb, mb, gs, ls, ss) in enumerate(bufs):
            pltpu.async_copy(h_hbm.at[src_v.at[b]], hb, gs)
            pltpu.async_copy(wrows(b), wb, ls)

        def body(i, carry):
            for b, (hb, wb, mb, gs, ls, ss) in enumerate(bufs):
                j = 2 * i + b
                pltpu.make_async_copy(h_hbm.at[src_v.at[j]], hb, gs).wait()
                pltpu.make_async_copy(wrows(j), wb, ls).wait()

                @pl.when(i > 0)
                def _():
                    pltpu.make_async_copy(
                        mb, agg_sh.at[dst_v.at[j - 2]], ss).wait()

                def inner(e2, icarry):
                    for u in range(2):
                        e = e2 * 2 + u
                        for q in range(HD // 16):
                            sl = pl.ds(q * 16, 16)
                            mb[e, sl] = hb[e, sl] * wb[e, sl]
                    return icarry

                lax.fori_loop(0, CHUNK // 2, inner, 0)

                @pl.when(i < NITER - 1)
                def _():
                    pltpu.async_copy(h_hbm.at[src_v.at[j + 2]], hb, gs)
                    pltpu.async_copy(wrows(j + 2), wb, ls)

                pltpu.async_copy(mb, agg_sh.at[dst_v.at[j]], ss, add=True)
            return carry

        lax.fori_loop(0, NITER, body, 0)
        for b, (hb, wb, mb, gs, ls, ss) in enumerate(bufs):
            pltpu.make_async_copy(
                mb, agg_sh.at[dst_v.at[CH - 2 + b]], ss).wait()
        plsc.subcore_barrier()
        pltpu.sync_copy(agg_sh.at[pl.ds(rbase, ROWS_PT)],
                        out_hbm.at[pl.ds(rbase, ROWS_PT)])

    @pl.when(c == 0)
    def _():
        run(hA_hbm, wA_hbm, outA)

    @pl.when(c == 1)
    def _():
        run(hB_hbm, wB_hbm, outB)


_edge_layer = functools.partial(
    pl.kernel,
    out_type=(
        jax.ShapeDtypeStruct((N_PAD, HD), jnp.float32),
        jax.ShapeDtypeStruct((N_PAD, HD), jnp.float32),
    ),
    mesh=_mesh,
    compiler_params=_sc_params,
    scratch_types=[
        pltpu.VMEM((CH, CHUNK), jnp.int32),
        pltpu.VMEM((CH, CHUNK), jnp.int32),
        pltpu.VMEM((CHUNK, HD), jnp.float32),
        pltpu.VMEM((CHUNK, HD), jnp.float32),
        pltpu.VMEM((CHUNK, HD), jnp.float32),
        pltpu.VMEM((CHUNK, HD), jnp.float32),
        pltpu.VMEM((CHUNK, HD), jnp.float32),
        pltpu.VMEM((CHUNK, HD), jnp.float32),
        pltpu.SemaphoreType.DMA,
        pltpu.SemaphoreType.DMA,
        pltpu.SemaphoreType.DMA,
        pltpu.SemaphoreType.DMA,
        pltpu.SemaphoreType.DMA,
        pltpu.SemaphoreType.DMA,
        pltpu.VMEM_SHARED((N_PAD, HD), jnp.float32),
    ],
)(_edge_layer_body)


BN = 1024


def _node_update_body(pA_ref, pB_ref, h_ref, Wu_ref, bu_ref, out_ref):
    agg = jnp.concatenate([pA_ref[...], pB_ref[...]], axis=1)
    pre = jnp.dot(agg, Wu_ref[...], preferred_element_type=jnp.float32)
    pre = pre + bu_ref[...]
    out_ref[...] = h_ref[...] + pre * jax.nn.sigmoid(pre)


def _node_update(pA, pB, h, Wu, bu2d):
    return pl.pallas_call(
        _node_update_body,
        grid=(N_PAD // BN,),
        in_specs=[
            pl.BlockSpec((BN, HD), lambda i: (i, 0)),
            pl.BlockSpec((BN, HD), lambda i: (i, 0)),
            pl.BlockSpec((BN, D), lambda i: (i, 0)),
            pl.BlockSpec((D, D), lambda i: (0, 0)),
            pl.BlockSpec((1, D), lambda i: (0, 0)),
        ],
        out_specs=pl.BlockSpec((BN, D), lambda i: (i, 0)),
        out_shape=jax.ShapeDtypeStruct((N_PAD, D), jnp.float32),
    )(pA, pB, h, Wu, bu2d)


def kernel(r, X, W_e1, W_u1, b_u1, W_e2, W_u2, b_u2, edge_index, spin_idx):
    src = edge_index[0].astype(jnp.int32)
    dst = edge_index[1].astype(jnp.int32)
    pad = E_PAD - E
    srcp = jnp.concatenate([src, jnp.zeros((pad,), jnp.int32)])
    dstp = jnp.concatenate([dst, jnp.zeros((pad,), jnp.int32)])
    rT = r.T.reshape(3 * N)

    d2 = _dist2(rT, srcp, dstp)
    d2_3d = d2.reshape(GB, 1, BE)
    w1a, w1b = _edge_filters(d2_3d, W_e1)
    w2a, w2b = _edge_filters(d2_3d, W_e2)
    spinp = jnp.concatenate(
        [spin_idx.astype(jnp.int32), jnp.zeros((N_PAD - N,), jnp.int32)])
    h0 = _h0(spinp.reshape(N_PAD, 1), X)

    src2 = srcp.reshape(E_PAD // CHUNK, CHUNK)
    dst2 = dstp.reshape(E_PAD // CHUNK, CHUNK)
    zeros = jnp.zeros((N_PAD, HD), jnp.float32)

    pA, pB = _edge_layer(h0[:, :HD], h0[:, HD:], w1a, w1b, src2, dst2, zeros)
    h1 = _node_update(pA, pB, h0, W_u1, b_u1.reshape(1, D))
    pA2, pB2 = _edge_layer(h1[:, :HD], h1[:, HD:], w2a, w2b, src2, dst2, zeros)
    h2 = _node_update(pA2, pB2, h1, W_u2, b_u2.reshape(1, D))
    return h2[:N]


# speedup vs baseline: 1.0877x; 1.0877x over previous
import functools

import jax
import jax.numpy as jnp
from jax import lax
from jax.experimental import pallas as pl
from jax.experimental.pallas import tpu as pltpu
from jax.experimental.pallas import tpu_sc as plsc

N = 10000
E = 320000
D = 128
K = 16
CUTOFF = 10.0
SIGMA = 0.5

NC = 2
NS = 16
NW = NC * NS
HD = D // 2
CHUNK = 128
CH = 160
T_E = CH * CHUNK
EPT = T_E // 2
E_PAD = T_E * NS
N_PAD = 10240
ROWS_PT = N_PAD // NS

_mesh = plsc.VectorSubcoreMesh(core_axis_name="c", subcore_axis_name="s")
_sc_params = pltpu.CompilerParams(
    needs_layout_passes=False, use_tc_tiling_on_sc=False)


def _dist2_body(rT_hbm, src_hbm, dst_hbm, d2_hbm, rT_v, si_v, di_v, out_v):
    c = lax.axis_index("c")
    s = lax.axis_index("s")
    wid = c * NS + s
    base = wid * EPT
    pltpu.sync_copy(rT_hbm, rT_v)
    pltpu.sync_copy(src_hbm.at[pl.ds(base, EPT)], si_v)
    pltpu.sync_copy(dst_hbm.at[pl.ds(base, EPT)], di_v)

    def body(j, carry):
        si = si_v[pl.ds(j * 16, 16)]
        di = di_v[pl.ds(j * 16, 16)]
        dx = plsc.load_gather(rT_v, [si]) - plsc.load_gather(rT_v, [di])
        dy = plsc.load_gather(rT_v, [si + N]) - plsc.load_gather(rT_v, [di + N])
        dz = plsc.load_gather(rT_v, [si + 2 * N]) - plsc.load_gather(rT_v, [di + 2 * N])
        out_v[pl.ds(j * 16, 16)] = dx * dx + dy * dy + dz * dz
        return carry

    lax.fori_loop(0, EPT // 16, body, 0)
    pltpu.sync_copy(out_v, d2_hbm.at[pl.ds(base, EPT)])


_dist2 = functools.partial(
    pl.kernel,
    out_type=jax.ShapeDtypeStruct((E_PAD,), jnp.float32),
    mesh=_mesh,
    compiler_params=_sc_params,
    scratch_types=[
        pltpu.VMEM((3 * N,), jnp.float32),
        pltpu.VMEM((EPT,), jnp.int32),
        pltpu.VMEM((EPT,), jnp.int32),
        pltpu.VMEM((EPT,), jnp.float32),
    ],
)(_dist2_body)


BE = 1024
GB = E_PAD // BE


def _edge_filters_body(d2_ref, We1_ref, We2_ref,
                       w1a_ref, w1b_ref, w2a_ref, w2b_ref):
    i = pl.program_id(0)
    d2 = jnp.reshape(d2_ref[...], (1, BE))
    dist = jnp.sqrt(d2 + 1e-12)
    t = jnp.clip(dist * (1.0 / CUTOFF), 0.0, 1.0)
    env = 0.5 * (jnp.cos(jnp.float32(3.14159265358979323846) * t) + 1.0)
    rows = i * BE + lax.broadcasted_iota(jnp.int32, (1, BE), 1)
    envm = jnp.where(rows < E, env, 0.0)
    inv = 1.0 / (2.0 * SIGMA * SIGMA)
    cols = []
    for k in range(K):
        mu_k = CUTOFF * k / (K - 1)
        cols.append(jnp.exp((dist - mu_k) * (dist - mu_k) * (-inv)) * envm)
    feat = jnp.concatenate(cols, axis=0)
    dn = (((0,), (0,)), ((), ()))
    w1 = lax.dot_general(feat, We1_ref[...], dn,
                         preferred_element_type=jnp.float32)
    w2 = lax.dot_general(feat, We2_ref[...], dn,
                         preferred_element_type=jnp.float32)
    w1a_ref[...] = w1[:, :HD]
    w1b_ref[...] = w1[:, HD:]
    w2a_ref[...] = w2[:, :HD]
    w2b_ref[...] = w2[:, HD:]


def _edge_filters(d2_3d, We1, We2):
    return pl.pallas_call(
        _edge_filters_body,
        grid=(GB,),
        in_specs=[
            pl.BlockSpec((1, 1, BE), lambda i: (i, 0, 0)),
            pl.BlockSpec((K, D), lambda i: (0, 0)),
            pl.BlockSpec((K, D), lambda i: (0, 0)),
        ],
        out_specs=[
            pl.BlockSpec((BE, HD), lambda i: (i, 0)),
            pl.BlockSpec((BE, HD), lambda i: (i, 0)),
            pl.BlockSpec((BE, HD), lambda i: (i, 0)),
            pl.BlockSpec((BE, HD), lambda i: (i, 0)),
        ],
        out_shape=[
            jax.ShapeDtypeStruct((E_PAD, HD), jnp.float32),
            jax.ShapeDtypeStruct((E_PAD, HD), jnp.float32),
            jax.ShapeDtypeStruct((E_PAD, HD), jnp.float32),
            jax.ShapeDtypeStruct((E_PAD, HD), jnp.float32),
        ],
    )(d2_3d, We1, We2)


def _h0_body(spin_ref, X_ref, out_ref):
    sp = spin_ref[...]
    out_ref[...] = jnp.where(sp == 0, X_ref[0:1, :], X_ref[1:2, :])


def _h0(spin2d, X):
    return pl.pallas_call(
        _h0_body,
        out_shape=jax.ShapeDtypeStruct((N_PAD, D), jnp.float32),
    )(spin2d, X)


NITER = CH // 2


def _edge_layer_body(hA_hbm, hB_hbm, wA_hbm, wB_hbm, src2_hbm, dst2_hbm,
                     zeros_hbm, outA, outB,
                     src_v, dst_v, h0_v, h1_v, w0_v, w1_v, m0_v, m1_v,
                     g0_s, g1_s, l0_s, l1_s, s0_s, s1_s, agg_sh):
    c = lax.axis_index("c")
    s = lax.axis_index("s")
    rbase = s * ROWS_PT
    pltpu.sync_copy(zeros_hbm.at[pl.ds(rbase, ROWS_PT)],
                    agg_sh.at[pl.ds(rbase, ROWS_PT)])
    pltpu.sync_copy(src2_hbm.at[pl.ds(s * CH, CH)], src_v)
    pltpu.sync_copy(dst2_hbm.at[pl.ds(s * CH, CH)], dst_v)
    plsc.subcore_barrier()

    bufs = ((h0_v, w0_v, m0_v, g0_s, l0_s, s0_s),
            (h1_v, w1_v, m1_v, g1_s, l1_s, s1_s))

    def run(h_hbm, w_hbm, out_hbm):
        def wrows(j):
            return w_hbm.at[pl.ds(s * T_E + j * CHUNK, CHUNK)]

        for b, (hb, wb, mb, gs, ls, ss) in enumerate(bufs):
            pltpu.async_copy(h_hbm.at[src_v.at[b]], hb, gs)
            pltpu.async_copy(wrows(b), wb, ls)

        def body(i, carry):
            for b, (hb, wb, mb, gs, ls, ss) in enumerate(bufs):
                j = 2 * i + b
                pltpu.make_async_copy(h_hbm.at[src_v.at[j]], hb, gs).wait()
                pltpu.make_async_copy(wrows(j), wb, ls).wait()

                @pl.when(i > 0)
                def _():
                    pltpu.make_async_copy(
                        mb, agg_sh.at[dst_v.at[j - 2]], ss).wait()

                def inner(e2, icarry):
                    for u in range(2):
                        e = e2 * 2 + u
                        for q in range(HD // 16):
                            sl = pl.ds(q * 16, 16)
                            mb[e, sl] = hb[e, sl] * wb[e, sl]
                    return icarry

                lax.fori_loop(0, CHUNK // 2, inner, 0)

                @pl.when(i < NITER - 1)
                def _():
                    pltpu.async_copy(h_hbm.at[src_v.at[j + 2]], hb, gs)
                    pltpu.async_copy(wrows(j + 2), wb, ls)

                pltpu.async_copy(mb, agg_sh.at[dst_v.at[j]], ss, add=True)
            return carry

        lax.fori_loop(0, NITER, body, 0)
        for b, (hb, wb, mb, gs, ls, ss) in enumerate(bufs):
            pltpu.make_async_copy(
                mb, agg_sh.at[dst_v.at[CH - 2 + b]], ss).wait()
        plsc.subcore_barrier()
        pltpu.sync_copy(agg_sh.at[pl.ds(rbase, ROWS_PT)],
                        out_hbm.at[pl.ds(rbase, ROWS_PT)])

    @pl.when(c == 0)
    def _():
        run(hA_hbm, wA_hbm, outA)

    @pl.when(c == 1)
    def _():
        run(hB_hbm, wB_hbm, outB)


_edge_layer = functools.partial(
    pl.kernel,
    out_type=(
        jax.ShapeDtypeStruct((N_PAD, HD), jnp.float32),
        jax.ShapeDtypeStruct((N_PAD, HD), jnp.float32),
    ),
    mesh=_mesh,
    compiler_params=_sc_params,
    scratch_types=[
        pltpu.VMEM((CH, CHUNK), jnp.int32),
        pltpu.VMEM((CH, CHUNK), jnp.int32),
        pltpu.VMEM((CHUNK, HD), jnp.float32),
        pltpu.VMEM((CHUNK, HD), jnp.float32),
        pltpu.VMEM((CHUNK, HD), jnp.float32),
        pltpu.VMEM((CHUNK, HD), jnp.float32),
        pltpu.VMEM((CHUNK, HD), jnp.float32),
        pltpu.VMEM((CHUNK, HD), jnp.float32),
        pltpu.SemaphoreType.DMA,
        pltpu.SemaphoreType.DMA,
        pltpu.SemaphoreType.DMA,
        pltpu.SemaphoreType.DMA,
        pltpu.SemaphoreType.DMA,
        pltpu.SemaphoreType.DMA,
        pltpu.VMEM_SHARED((N_PAD, HD), jnp.float32),
    ],
)(_edge_layer_body)


BN = 1024


def _node_update_body(pA_ref, pB_ref, h_ref, Wu_ref, bu_ref, out_ref):
    agg = jnp.concatenate([pA_ref[...], pB_ref[...]], axis=1)
    pre = jnp.dot(agg, Wu_ref[...], preferred_element_type=jnp.float32)
    pre = pre + bu_ref[...]
    out_ref[...] = h_ref[...] + pre * jax.nn.sigmoid(pre)


def _node_update(pA, pB, h, Wu, bu2d):
    return pl.pallas_call(
        _node_update_body,
        grid=(N_PAD // BN,),
        in_specs=[
            pl.BlockSpec((BN, HD), lambda i: (i, 0)),
            pl.BlockSpec((BN, HD), lambda i: (i, 0)),
            pl.BlockSpec((BN, D), lambda i: (i, 0)),
            pl.BlockSpec((D, D), lambda i: (0, 0)),
            pl.BlockSpec((1, D), lambda i: (0, 0)),
        ],
        out_specs=pl.BlockSpec((BN, D), lambda i: (i, 0)),
        out_shape=jax.ShapeDtypeStruct((N_PAD, D), jnp.float32),
    )(pA, pB, h, Wu, bu2d)


def kernel(r, X, W_e1, W_u1, b_u1, W_e2, W_u2, b_u2, edge_index, spin_idx):
    src = edge_index[0].astype(jnp.int32)
    dst = edge_index[1].astype(jnp.int32)
    pad = E_PAD - E
    srcp = jnp.concatenate([src, jnp.zeros((pad,), jnp.int32)])
    dstp = jnp.concatenate([dst, jnp.zeros((pad,), jnp.int32)])
    rT = r.T.reshape(3 * N)

    d2 = _dist2(rT, srcp, dstp)
    w1a, w1b, w2a, w2b = _edge_filters(d2.reshape(GB, 1, BE), W_e1, W_e2)
    spinp = jnp.concatenate(
        [spin_idx.astype(jnp.int32), jnp.zeros((N_PAD - N,), jnp.int32)])
    h0 = _h0(spinp.reshape(N_PAD, 1), X)

    src2 = srcp.reshape(E_PAD // CHUNK, CHUNK)
    dst2 = dstp.reshape(E_PAD // CHUNK, CHUNK)
    zeros = jnp.zeros((N_PAD, HD), jnp.float32)

    pA, pB = _edge_layer(h0[:, :HD], h0[:, HD:], w1a, w1b, src2, dst2, zeros)
    h1 = _node_update(pA, pB, h0, W_u1, b_u1.reshape(1, D))
    pA2, pB2 = _edge_layer(h1[:, :HD], h1[:, HD:], w2a, w2b, src2, dst2, zeros)
    h2 = _node_update(pA2, pB2, h1, W_u2, b_u2.reshape(1, D))
    return h2[:N]


# speedup vs baseline: 1.4999x; 1.3790x over previous
import functools

import jax
import jax.numpy as jnp
from jax import lax
from jax.experimental import pallas as pl
from jax.experimental.pallas import tpu as pltpu
from jax.experimental.pallas import tpu_sc as plsc

N = 10000
E = 320000
D = 128
K = 16
CUTOFF = 10.0
SIGMA = 0.5

NC = 2
NS = 16
NW = NC * NS
HD = D // 2
CHUNK = 128
CH = 160
T_E = CH * CHUNK
EPT = T_E // 2
E_PAD = T_E * NS
N_PAD = 10240
ROWS_PT = N_PAD // NS

_mesh = plsc.VectorSubcoreMesh(core_axis_name="c", subcore_axis_name="s")
_sc_params = pltpu.CompilerParams(
    needs_layout_passes=False, use_tc_tiling_on_sc=False)


def _dist2_body(rT_hbm, src_hbm, dst_hbm, d2_hbm, rT_v, si_v, di_v, out_v):
    c = lax.axis_index("c")
    s = lax.axis_index("s")
    wid = c * NS + s
    base = wid * EPT
    pltpu.sync_copy(rT_hbm, rT_v)
    pltpu.sync_copy(src_hbm.at[pl.ds(base, EPT)], si_v)
    pltpu.sync_copy(dst_hbm.at[pl.ds(base, EPT)], di_v)

    def body(j, carry):
        si = si_v[pl.ds(j * 16, 16)]
        di = di_v[pl.ds(j * 16, 16)]
        dx = plsc.load_gather(rT_v, [si]) - plsc.load_gather(rT_v, [di])
        dy = plsc.load_gather(rT_v, [si + N]) - plsc.load_gather(rT_v, [di + N])
        dz = plsc.load_gather(rT_v, [si + 2 * N]) - plsc.load_gather(rT_v, [di + 2 * N])
        out_v[pl.ds(j * 16, 16)] = dx * dx + dy * dy + dz * dz
        return carry

    lax.fori_loop(0, EPT // 16, body, 0)
    pltpu.sync_copy(out_v, d2_hbm.at[pl.ds(base, EPT)])


_dist2 = functools.partial(
    pl.kernel,
    out_type=jax.ShapeDtypeStruct((E_PAD,), jnp.float32),
    mesh=_mesh,
    compiler_params=_sc_params,
    scratch_types=[
        pltpu.VMEM((3 * N,), jnp.float32),
        pltpu.VMEM((EPT,), jnp.int32),
        pltpu.VMEM((EPT,), jnp.int32),
        pltpu.VMEM((EPT,), jnp.float32),
    ],
)(_dist2_body)


BE = 1024
GB = E_PAD // BE


def _edge_filters_body(d2_ref, We1_ref, We2_ref, w1_ref, w2_ref):
    i = pl.program_id(0)
    d2 = jnp.reshape(d2_ref[...], (1, BE))
    dist = jnp.sqrt(d2 + 1e-12)
    t = jnp.clip(dist * (1.0 / CUTOFF), 0.0, 1.0)
    env = 0.5 * (jnp.cos(jnp.float32(3.14159265358979323846) * t) + 1.0)
    rows = i * BE + lax.broadcasted_iota(jnp.int32, (1, BE), 1)
    envm = jnp.where(rows < E, env, 0.0)
    inv = 1.0 / (2.0 * SIGMA * SIGMA)
    cols = []
    for k in range(K):
        mu_k = CUTOFF * k / (K - 1)
        cols.append(jnp.exp((dist - mu_k) * (dist - mu_k) * (-inv)) * envm)
    feat = jnp.concatenate(cols, axis=0)
    dn = (((0,), (0,)), ((), ()))
    w1_ref[...] = lax.dot_general(feat, We1_ref[...], dn,
                                  preferred_element_type=jnp.float32)
    w2_ref[...] = lax.dot_general(feat, We2_ref[...], dn,
                                  preferred_element_type=jnp.float32)


def _edge_filters(d2_3d, We1, We2):
    return pl.pallas_call(
        _edge_filters_body,
        grid=(GB,),
        in_specs=[
            pl.BlockSpec((1, 1, BE), lambda i: (i, 0, 0)),
            pl.BlockSpec((K, D), lambda i: (0, 0)),
            pl.BlockSpec((K, D), lambda i: (0, 0)),
        ],
        out_specs=[
            pl.BlockSpec((BE, D), lambda i: (i, 0)),
            pl.BlockSpec((BE, D), lambda i: (i, 0)),
        ],
        out_shape=[
            jax.ShapeDtypeStruct((E_PAD, D), jnp.float32),
            jax.ShapeDtypeStruct((E_PAD, D), jnp.float32),
        ],
    )(d2_3d, We1, We2)


def _h0_body(spin_ref, X_ref, out_ref):
    sp = spin_ref[...]
    out_ref[...] = jnp.where(sp == 0, X_ref[0:1, :], X_ref[1:2, :])


def _h0(spin2d, X):
    return pl.pallas_call(
        _h0_body,
        out_shape=jax.ShapeDtypeStruct((N_PAD, D), jnp.float32),
    )(spin2d, X)


NITER = CH // 2


def _edge_layer_body(hA_hbm, hB_hbm, w_hbm, src2_hbm, dst2_hbm,
                     zeros_hbm, outA, outB,
                     src_v, dst_v, h0_v, h1_v, w0_v, w1_v, m0_v, m1_v,
                     g0_s, g1_s, l0_s, l1_s, s0_s, s1_s, agg_sh):
    c = lax.axis_index("c")
    s = lax.axis_index("s")
    col = c * HD
    rbase = s * ROWS_PT
    pltpu.sync_copy(zeros_hbm.at[pl.ds(rbase, ROWS_PT)],
                    agg_sh.at[pl.ds(rbase, ROWS_PT)])
    pltpu.sync_copy(src2_hbm.at[pl.ds(s * CH, CH)], src_v)
    pltpu.sync_copy(dst2_hbm.at[pl.ds(s * CH, CH)], dst_v)
    plsc.subcore_barrier()

    bufs = ((h0_v, w0_v, m0_v, g0_s, l0_s, s0_s),
            (h1_v, w1_v, m1_v, g1_s, l1_s, s1_s))

    def run(h_hbm, out_hbm):
        def wrows(j):
            return w_hbm.at[pl.ds(s * T_E + j * CHUNK, CHUNK),
                            pl.ds(col, HD)]

        for b, (hb, wb, mb, gs, ls, ss) in enumerate(bufs):
            pltpu.async_copy(h_hbm.at[src_v.at[b]], hb, gs)
            pltpu.async_copy(wrows(b), wb, ls)

        def body(i, carry):
            for b, (hb, wb, mb, gs, ls, ss) in enumerate(bufs):
                j = 2 * i + b
                pltpu.make_async_copy(h_hbm.at[src_v.at[j]], hb, gs).wait()
                pltpu.make_async_copy(wrows(j), wb, ls).wait()

                @pl.when(i > 0)
                def _():
                    pltpu.make_async_copy(
                        mb, agg_sh.at[dst_v.at[j - 2]], ss).wait()

                def inner(e2, icarry):
                    for u in range(2):
                        e = e2 * 2 + u
                        for q in range(HD // 16):
                            sl = pl.ds(q * 16, 16)
                            mb[e, sl] = hb[e, sl] * wb[e, sl]
                    return icarry

                lax.fori_loop(0, CHUNK // 2, inner, 0)

                @pl.when(i < NITER - 1)
                def _():
                    pltpu.async_copy(h_hbm.at[src_v.at[j + 2]], hb, gs)
                    pltpu.async_copy(wrows(j + 2), wb, ls)

                pltpu.async_copy(mb, agg_sh.at[dst_v.at[j]], ss, add=True)
            return carry

        lax.fori_loop(0, NITER, body, 0)
        for b, (hb, wb, mb, gs, ls, ss) in enumerate(bufs):
            pltpu.make_async_copy(
                mb, agg_sh.at[dst_v.at[CH - 2 + b]], ss).wait()
        plsc.subcore_barrier()
        pltpu.sync_copy(agg_sh.at[pl.ds(rbase, ROWS_PT)],
                        out_hbm.at[pl.ds(rbase, ROWS_PT)])

    @pl.when(c == 0)
    def _():
        run(hA_hbm, outA)

    @pl.when(c == 1)
    def _():
        run(hB_hbm, outB)


_edge_layer = functools.partial(
    pl.kernel,
    out_type=(
        jax.ShapeDtypeStruct((N_PAD, HD), jnp.float32),
        jax.ShapeDtypeStruct((N_PAD, HD), jnp.float32),
    ),
    mesh=_mesh,
    compiler_params=_sc_params,
    scratch_types=[
        pltpu.VMEM((CH, CHUNK), jnp.int32),
        pltpu.VMEM((CH, CHUNK), jnp.int32),
        pltpu.VMEM((CHUNK, HD), jnp.float32),
        pltpu.VMEM((CHUNK, HD), jnp.float32),
        pltpu.VMEM((CHUNK, HD), jnp.float32),
        pltpu.VMEM((CHUNK, HD), jnp.float32),
        pltpu.VMEM((CHUNK, HD), jnp.float32),
        pltpu.VMEM((CHUNK, HD), jnp.float32),
        pltpu.SemaphoreType.DMA,
        pltpu.SemaphoreType.DMA,
        pltpu.SemaphoreType.DMA,
        pltpu.SemaphoreType.DMA,
        pltpu.SemaphoreType.DMA,
        pltpu.SemaphoreType.DMA,
        pltpu.VMEM_SHARED((N_PAD, HD), jnp.float32),
    ],
)(_edge_layer_body)


BN = 1024


def _node_update_body(pA_ref, pB_ref, h_ref, Wu_ref, bu_ref, out_ref):
    agg = jnp.concatenate([pA_ref[...], pB_ref[...]], axis=1)
    pre = jnp.dot(agg, Wu_ref[...], preferred_element_type=jnp.float32)
    pre = pre + bu_ref[...]
    out_ref[...] = h_ref[...] + pre * jax.nn.sigmoid(pre)


def _node_update(pA, pB, h, Wu, bu2d):
    return pl.pallas_call(
        _node_update_body,
        grid=(N_PAD // BN,),
        in_specs=[
            pl.BlockSpec((BN, HD), lambda i: (i, 0)),
            pl.BlockSpec((BN, HD), lambda i: (i, 0)),
            pl.BlockSpec((BN, D), lambda i: (i, 0)),
            pl.BlockSpec((D, D), lambda i: (0, 0)),
            pl.BlockSpec((1, D), lambda i: (0, 0)),
        ],
        out_specs=pl.BlockSpec((BN, D), lambda i: (i, 0)),
        out_shape=jax.ShapeDtypeStruct((N_PAD, D), jnp.float32),
    )(pA, pB, h, Wu, bu2d)


def kernel(r, X, W_e1, W_u1, b_u1, W_e2, W_u2, b_u2, edge_index, spin_idx):
    src = edge_index[0].astype(jnp.int32)
    dst = edge_index[1].astype(jnp.int32)
    pad = E_PAD - E
    srcp = jnp.concatenate([src, jnp.zeros((pad,), jnp.int32)])
    dstp = jnp.concatenate([dst, jnp.zeros((pad,), jnp.int32)])
    rT = r.T.reshape(3 * N)

    d2 = _dist2(rT, srcp, dstp)
    w1, w2 = _edge_filters(d2.reshape(GB, 1, BE), W_e1, W_e2)
    spinp = jnp.concatenate(
        [spin_idx.astype(jnp.int32), jnp.zeros((N_PAD - N,), jnp.int32)])
    h0 = _h0(spinp.reshape(N_PAD, 1), X)

    src2 = srcp.reshape(E_PAD // CHUNK, CHUNK)
    dst2 = dstp.reshape(E_PAD // CHUNK, CHUNK)
    zeros = jnp.zeros((N_PAD, HD), jnp.float32)

    pA, pB = _edge_layer(h0[:, :HD], h0[:, HD:], w1, src2, dst2, zeros)
    h1 = _node_update(pA, pB, h0, W_u1, b_u1.reshape(1, D))
    pA2, pB2 = _edge_layer(h1[:, :HD], h1[:, HD:], w2, src2, dst2, zeros)
    h2 = _node_update(pA2, pB2, h1, W_u2, b_u2.reshape(1, D))
    return h2[:N]
